# baseline (speedup 1.0000x reference)
import jax
import jax.numpy as jnp
from jax import lax
from jax.experimental import pallas as pl

N_CHUNK = 512


def _body(nodes_ref, w_ref, out_ref, *, steps_per_row):
    j = pl.program_id(0)

    w = w_ref[...]
    x = nodes_ref[...]
    part = jnp.sum(x * (w * (1.0 / 4096.0)), axis=0)

    @pl.when(j % steps_per_row == 0)
    def _():
        out_ref[...] = jnp.zeros_like(out_ref)

    out_ref[0, 0, :] += part


def kernel(nodes, weights):
    B, N, F = nodes.shape
    nodes2 = nodes.reshape(B * N, F)
    w2 = weights.reshape(B * N, 1)
    steps_per_row = N // N_CHUNK
    grid = (B * steps_per_row,)
    import functools
    out = pl.pallas_call(
        functools.partial(_body, steps_per_row=steps_per_row),
        grid=grid,
        in_specs=[
            pl.BlockSpec((N_CHUNK, F), lambda j: (j, 0)),
            pl.BlockSpec((N_CHUNK, 1), lambda j: (j, 0)),
        ],
        out_specs=pl.BlockSpec(
            (1, 1, F), lambda j, spr=steps_per_row: (j // spr, 0, 0)
        ),
        out_shape=jax.ShapeDtypeStruct((B, 1, F), jnp.float32),
    )(nodes2, w2)
    return out.reshape(B, F)


# speedup vs baseline: 1.0602x; 1.0602x over previous
import functools

import jax
import jax.numpy as jnp
from jax import lax
from jax.experimental import pallas as pl

N_CHUNK = 2048


def _body(nodes_ref, w_ref, out_ref, *, steps_per_row):
    j = pl.program_id(0)

    w = w_ref[pl.ds(0, N_CHUNK), :]
    x = nodes_ref[...]
    part = jnp.sum(x * (w * (1.0 / 4096.0)), axis=0)

    @pl.when(j % steps_per_row == 0)
    def _():
        out_ref[...] = jnp.zeros_like(out_ref)

    out_ref[0, 0, :] += part


def kernel(nodes, weights):
    B, N, F = nodes.shape
    nodes2 = nodes.reshape(B * N, F)
    w2 = weights.reshape(B * N, 1)
    steps_per_row = N // N_CHUNK
    grid = (B * steps_per_row,)
    out = pl.pallas_call(
        functools.partial(_body, steps_per_row=steps_per_row),
        grid=grid,
        in_specs=[
            pl.BlockSpec((N_CHUNK, F), lambda j: (j, 0)),
            pl.BlockSpec((N_CHUNK, 1), lambda j: (j, 0)),
        ],
        out_specs=pl.BlockSpec(
            (1, 1, F), lambda j, spr=steps_per_row: (j // spr, 0, 0)
        ),
        out_shape=jax.ShapeDtypeStruct((B, 1, F), jnp.float32),
    )(nodes2, w2)
    return out.reshape(B, F)


# speedup vs baseline: 1.0722x; 1.0113x over previous
import functools

import jax
import jax.numpy as jnp
from jax import lax
from jax.experimental import pallas as pl
from jax.experimental.pallas import tpu as pltpu

CHUNK = 256
NBUF = 8


def _chunk_copy(nodes_hbm, buf, sems, c, slot):
    return pltpu.make_async_copy(
        nodes_hbm.at[pl.ds(c * CHUNK, CHUNK), :],
        buf.at[slot],
        sems.at[slot],
    )


def _body(nodes_hbm, w_ref, out_ref, buf, sems, *, steps_per_row):
    b = pl.program_id(0)
    j = pl.program_id(1)
    nsteps = pl.num_programs(0) * pl.num_programs(1)
    c = b * steps_per_row + j

    @pl.when(c == 0)
    def _():
        for k in range(NBUF):
            _chunk_copy(nodes_hbm, buf, sems, k, k).start()

    slot = lax.rem(c, NBUF)
    _chunk_copy(nodes_hbm, buf, sems, c, slot).wait()

    w = w_ref[pl.ds(c * CHUNK, CHUNK), :] * (1.0 / 4096.0)
    part = jnp.sum(buf[slot] * w, axis=0)

    @pl.when(j == 0)
    def _():
        out_ref[...] = jnp.zeros_like(out_ref)

    out_ref[0, 0, :] += part

    nxt = c + NBUF

    @pl.when(nxt < nsteps)
    def _():
        _chunk_copy(nodes_hbm, buf, sems, nxt, lax.rem(nxt, NBUF)).start()


def kernel(nodes, weights):
    B, N, F = nodes.shape
    nodes2 = nodes.reshape(B * N, F)
    w2 = weights.reshape(B * N, 1)
    steps_per_row = N // CHUNK
    grid = (B, steps_per_row)
    out = pl.pallas_call(
        functools.partial(_body, steps_per_row=steps_per_row),
        grid=grid,
        in_specs=[
            pl.BlockSpec(memory_space=pl.ANY),
            pl.BlockSpec((B * N, 1), lambda b, j: (0, 0)),
        ],
        out_specs=pl.BlockSpec((1, 1, F), lambda b, j: (b, 0, 0)),
        out_shape=jax.ShapeDtypeStruct((B, 1, F), jnp.float32),
        scratch_shapes=[
            pltpu.VMEM((NBUF, CHUNK, F), jnp.float32),
            pltpu.SemaphoreType.DMA((NBUF,)),
        ],
    )(nodes2, w2)
    return out.reshape(B, F)
